# baseline (speedup 1.0000x reference)
import jax
import jax.numpy as jnp
from jax import lax
from jax.experimental import pallas as pl
from jax.experimental.pallas import tpu as pltpu
from jax.experimental.pallas import tpu_sc as plsc

PI = 3.141592653589793
GAMMA = 9.0
EMB_RANGE = 0.34375

B = 1024
D = 32
H = D // 2
NT = 1000

_NC, _NS = 2, 16
_NW = _NC * _NS
_BPW = B // _NW


def _sc_gather_body(ent_hbm, etab_hbm, e_out, eidx_v, erows_v, sem):
    wid = lax.axis_index("s") * _NC + lax.axis_index("c")
    base = wid * _BPW
    pltpu.sync_copy(ent_hbm.at[pl.ds(base, _BPW)], eidx_v)
    pltpu.async_copy(etab_hbm.at[eidx_v], erows_v, sem).wait()
    pltpu.sync_copy(erows_v, e_out.at[pl.ds(base, _BPW)])


_sc_gather_cache = []


def _sc_gather(*a):
    if not _sc_gather_cache:
        _sc_gather_cache.append(pl.kernel(
            _sc_gather_body,
            out_type=jax.ShapeDtypeStruct((B, D), jnp.float32),
            mesh=plsc.VectorSubcoreMesh(core_axis_name="c",
                                        subcore_axis_name="s"),
            compiler_params=pltpu.CompilerParams(use_tc_tiling_on_sc=False),
            scratch_types=[
                pltpu.VMEM((_BPW,), jnp.int32),
                pltpu.VMEM((_BPW, D), jnp.float32),
                pltpu.SemaphoreType.DMA,
            ],
        ))
    return _sc_gather_cache[0](*a)


def _tc_gather_body(idx_ref, tab_ref, out_ref):
    i = pl.program_id(0)
    r = idx_ref[i] % 8
    out_ref[pl.ds(i % 8, 1), :] = tab_ref[pl.ds(r, 1), :]


_tc_gather = pl.pallas_call(
    _tc_gather_body,
    grid_spec=pltpu.PrefetchScalarGridSpec(
        num_scalar_prefetch=1,
        grid=(B,),
        in_specs=[pl.BlockSpec((8, D), lambda i, idx_ref: (idx_ref[i] // 8, 0))],
        out_specs=pl.BlockSpec((8, D), lambda i, idx_ref: (i // 8, 0)),
    ),
    out_shape=jax.ShapeDtypeStruct((B, D), jnp.float32),
)


_BI = 128


def _fused_body(e_ref, tt_ref, tix_ref, mw_ref, pw_ref, out_ref, mod_s, ph_s):
    i = pl.program_id(0)

    @pl.when(i == 0)
    def _():
        sel = (tix_ref[...] == lax.broadcasted_iota(jnp.int32, (B, NT), 1))
        t = lax.dot_general(sel.astype(jnp.float32), tt_ref[...],
                            (((1,), (0,)), ((), ())),
                            preferred_element_type=jnp.float32)
        e = e_ref[...]
        s = PI / EMB_RANGE
        dr = (e[:, :H] - t[:, :H]) * s
        di = (e[:, H:] - t[:, H:]) * s
        mod_s[...] = jnp.sqrt(dr * dr + di * di) * mw_ref[0] - GAMMA
        ph = jnp.sum(jnp.cos(dr) * jnp.cos(di), axis=1) * pw_ref[0]
        ph_s[...] = ph[None, :]

    mod_blk = mod_s[pl.ds(i * _BI, _BI), :]
    out_ref[...] = mod_blk[:, :, None] + ph_s[...][None, :, :]


_fused_call = pl.pallas_call(
    _fused_body,
    grid=(B // _BI,),
    in_specs=[
        pl.BlockSpec((B, D), lambda i: (0, 0)),
        pl.BlockSpec((NT, D), lambda i: (0, 0)),
        pl.BlockSpec((B, 1), lambda i: (0, 0)),
        pl.BlockSpec(memory_space=pltpu.SMEM),
        pl.BlockSpec(memory_space=pltpu.SMEM),
    ],
    out_specs=pl.BlockSpec((_BI, H, B), lambda i: (i, 0, 0)),
    out_shape=jax.ShapeDtypeStruct((B, H, B), jnp.float32),
    scratch_shapes=[
        pltpu.VMEM((B, H), jnp.float32),
        pltpu.VMEM((1, B), jnp.float32),
    ],
)


def kernel(ent, type_idx, ent_table, type_table, modulus_weight, phase_weight):
    ent = ent.astype(jnp.int32)
    tix_col = type_idx.astype(jnp.int32)[:, None]
    e = _tc_gather(ent, ent_table)
    out3 = _fused_call(e, type_table, tix_col, modulus_weight, phase_weight)
    return out3.transpose(0, 2, 1)


# speedup vs baseline: 4.4588x; 4.4588x over previous
import jax
import jax.numpy as jnp
from jax import lax
from jax.experimental import pallas as pl
from jax.experimental.pallas import tpu as pltpu

PI = 3.141592653589793
GAMMA = 9.0
EMB_RANGE = 0.34375

B = 1024
D = 32
H = D // 2
NT = 1000
_BI = 128


def _fused_body(idx_sref, tab_ref, entcol_ref, tt_ref, tix_ref, mw_ref,
                pw_ref, out_ref, tiles_s, mod_s, ph_s, sem):
    i = pl.program_id(0)

    @pl.when(i == 0)
    def _():
        def _fire(r, _):
            base = (idx_sref[r] // 8) * 8
            pltpu.make_async_copy(
                tab_ref.at[pl.ds(base, 8), :], tiles_s.at[r], sem).start()
            return 0

        lax.fori_loop(0, B, _fire, 0)

        sel = (tix_ref[...] == lax.broadcasted_iota(jnp.int32, (B, NT), 1))
        t = lax.dot_general(sel.astype(jnp.float32), tt_ref[...],
                            (((1,), (0,)), ((), ())),
                            preferred_element_type=jnp.float32)

        def _drain(r, _):
            pltpu.make_async_copy(
                tab_ref.at[pl.ds(0, 8), :], tiles_s.at[0], sem).wait()
            return 0

        lax.fori_loop(0, B, _drain, 0)

        sel8 = (entcol_ref[...] % 8 ==
                lax.broadcasted_iota(jnp.int32, (B, 8), 1)).astype(jnp.float32)
        e = jnp.sum(tiles_s[...] * lax.broadcast_in_dim(sel8, (B, 8, D), (0, 1)),
                    axis=1)

        s = PI / EMB_RANGE
        dr = (e[:, :H] - t[:, :H]) * s
        di = (e[:, H:] - t[:, H:]) * s
        mod_s[...] = jnp.sqrt(dr * dr + di * di) * mw_ref[0] - GAMMA
        ph = jnp.sum(jnp.cos(dr) * jnp.cos(di), axis=1) * pw_ref[0]
        ph_s[...] = ph[None, :]

    mod_blk = mod_s[pl.ds(i * _BI, _BI), :]
    out_ref[...] = mod_blk[:, :, None] + ph_s[...][None, :, :]


_fused_call = pl.pallas_call(
    _fused_body,
    grid_spec=pltpu.PrefetchScalarGridSpec(
        num_scalar_prefetch=1,
        grid=(B // _BI,),
        in_specs=[
            pl.BlockSpec(memory_space=pltpu.MemorySpace.HBM),
            pl.BlockSpec((B, 1), lambda i, idx_ref: (0, 0)),
            pl.BlockSpec((NT, D), lambda i, idx_ref: (0, 0)),
            pl.BlockSpec((B, 1), lambda i, idx_ref: (0, 0)),
            pl.BlockSpec(memory_space=pltpu.SMEM),
            pl.BlockSpec(memory_space=pltpu.SMEM),
        ],
        out_specs=pl.BlockSpec((_BI, H, B), lambda i, idx_ref: (i, 0, 0)),
        scratch_shapes=[
            pltpu.VMEM((B, 8, D), jnp.float32),
            pltpu.VMEM((B, H), jnp.float32),
            pltpu.VMEM((1, B), jnp.float32),
            pltpu.SemaphoreType.DMA,
        ],
    ),
    out_shape=jax.ShapeDtypeStruct((B, H, B), jnp.float32),
)


def kernel(ent, type_idx, ent_table, type_table, modulus_weight, phase_weight):
    ent = ent.astype(jnp.int32)
    tix_col = type_idx.astype(jnp.int32)[:, None]
    ent_col = ent[:, None]
    out3 = _fused_call(ent, ent_table, ent_col, type_table, tix_col,
                       modulus_weight, phase_weight)
    return out3.transpose(0, 2, 1)


# speedup vs baseline: 4.9952x; 1.1203x over previous
import jax
import jax.numpy as jnp
from jax import lax
from jax.experimental import pallas as pl
from jax.experimental.pallas import tpu as pltpu

PI = 3.141592653589793
GAMMA = 9.0
EMB_RANGE = 0.34375

B = 1024
D = 32
H = D // 2
NT = 1000
_BI = 128


def _fused_body(idx_sref, tab_ref, entcol_ref, tt_ref, tix_ref, mw_ref,
                pw_ref, out_ref, tiles_s, mod_s, ph_s, sem):
    i = pl.program_id(0)

    @pl.when(i == 0)
    def _():
        def _fire(g, _):
            for u in range(8):
                r = g * 8 + u
                base = (idx_sref[r] // 8) * 8
                pltpu.make_async_copy(
                    tab_ref.at[pl.ds(base, 8), :], tiles_s.at[r], sem).start()
            return 0

        lax.fori_loop(0, B // 8, _fire, 0)

        sel = (tix_ref[...] == lax.broadcasted_iota(jnp.int32, (B, NT), 1))
        t = lax.dot_general(sel.astype(jnp.float32), tt_ref[...],
                            (((1,), (0,)), ((), ())),
                            preferred_element_type=jnp.float32)

        def _drain(g, _):
            for _u in range(8):
                pltpu.make_async_copy(
                    tab_ref.at[pl.ds(0, 8), :], tiles_s.at[0], sem).wait()
            return 0

        lax.fori_loop(0, B // 8, _drain, 0)

        sel8 = (entcol_ref[...] % 8 ==
                lax.broadcasted_iota(jnp.int32, (B, 8), 1)).astype(jnp.float32)
        e = jnp.sum(tiles_s[...] * lax.broadcast_in_dim(sel8, (B, 8, D), (0, 1)),
                    axis=1)

        s = PI / EMB_RANGE
        dr = (e[:, :H] - t[:, :H]) * s
        di = (e[:, H:] - t[:, H:]) * s
        mod_s[...] = jnp.sqrt(dr * dr + di * di) * mw_ref[0] - GAMMA
        ph = jnp.sum(jnp.cos(dr) * jnp.cos(di), axis=1) * pw_ref[0]
        ph_s[...] = ph[None, :]

    mod_blk = mod_s[pl.ds(i * _BI, _BI), :]
    out_ref[...] = mod_blk[:, :, None] + ph_s[...][None, :, :]


_fused_call = pl.pallas_call(
    _fused_body,
    grid_spec=pltpu.PrefetchScalarGridSpec(
        num_scalar_prefetch=1,
        grid=(B // _BI,),
        in_specs=[
            pl.BlockSpec(memory_space=pltpu.MemorySpace.HBM),
            pl.BlockSpec((B, 1), lambda i, idx_ref: (0, 0)),
            pl.BlockSpec((NT, D), lambda i, idx_ref: (0, 0)),
            pl.BlockSpec((B, 1), lambda i, idx_ref: (0, 0)),
            pl.BlockSpec(memory_space=pltpu.SMEM),
            pl.BlockSpec(memory_space=pltpu.SMEM),
        ],
        out_specs=pl.BlockSpec((_BI, H, B), lambda i, idx_ref: (i, 0, 0)),
        scratch_shapes=[
            pltpu.VMEM((B, 8, D), jnp.float32),
            pltpu.VMEM((B, H), jnp.float32),
            pltpu.VMEM((1, B), jnp.float32),
            pltpu.SemaphoreType.DMA,
        ],
    ),
    out_shape=jax.ShapeDtypeStruct((B, H, B), jnp.float32),
)


def kernel(ent, type_idx, ent_table, type_table, modulus_weight, phase_weight):
    ent = ent.astype(jnp.int32)
    tix_col = type_idx.astype(jnp.int32)[:, None]
    ent_col = ent[:, None]
    out3 = _fused_call(ent, ent_table, ent_col, type_table, tix_col,
                       modulus_weight, phase_weight)
    return out3.transpose(0, 2, 1)
